# Initial kernel scaffold; baseline (speedup 1.0000x reference)
import jax
import jax.numpy as jnp
from jax.experimental import pallas as pl


def kernel(x, adjacency_matrix, W1, b1, W2, b2, ln_gamma, ln_beta):
    raise NotImplementedError("write your pallas kernel here")


# speedup vs baseline: 1675.7448x; 1675.7448x over previous
import jax
import jax.numpy as jnp
from jax.experimental import pallas as pl

_HI = jax.lax.Precision.HIGHEST


def _dot(a, b, dims, prec=jax.lax.Precision.DEFAULT):
    return jax.lax.dot_general(a, b, (dims, ((), ())), precision=prec,
                               preferred_element_type=jnp.float32)


def _graph_body(x_ref, adj_ref, w1_ref, b1_ref, w2_ref, b2_ref, g_ref, bt_ref,
                out_ref):
    adj = adj_ref[0]
    xi = x_ref[0]
    C = adj.shape[0]
    f32 = jnp.float32

    M = (adj != 0.0).astype(f32)
    Mb = M.astype(jnp.bfloat16)

    r_io = jax.lax.broadcasted_iota(jnp.int32, (C, C), 0)
    c_io = jax.lax.broadcasted_iota(jnp.int32, (C, C), 1)
    Ub = (r_io <= c_io).astype(jnp.bfloat16)
    Usf = (r_io < c_io).astype(f32)
    Uf = (r_io <= c_io).astype(f32)

    rc = jnp.sum(M, axis=1, keepdims=True)
    incl_lane = jnp.round(_dot(rc, Uf, ((0,), (0,)), _HI))
    roff_lane = jnp.round(_dot(rc, Usf, ((0,), (0,)), _HI))
    roff_col = jnp.round(_dot(Usf, rc, ((0,), (0,)), _HI))

    j_col = jax.lax.broadcasted_iota(jnp.int32, (C, 1), 0).astype(f32)
    R = jnp.logical_and(j_col >= roff_lane, j_col < incl_lane).astype(f32)
    Rb = R.astype(jnp.bfloat16)

    Mg = _dot(Rb, Mb, ((1,), (0,)))
    Wg = _dot(Mg.astype(jnp.bfloat16), Ub, ((1,), (0,)))
    Wgex = Wg - Mg
    Ag = _dot(R, adj, ((1,), (0,)))

    l_col = j_col - jnp.round(_dot(R, roff_col, ((1,), (0,)), _HI))
    Colh = jnp.logical_and(Wg >= l_col + 1.0, Wgex <= l_col).astype(f32)
    ew_col = jnp.sum(Ag * Colh, axis=1, keepdims=True)

    ones_col = jnp.ones((C, 1), f32)
    Be = jnp.round(_dot(M, ones_col, ((0,), (0,)), _HI))
    Binv = jnp.where(Be > 0, 1.0 / Be, 0.0)
    Dn = _dot(M, ew_col, ((1,), (0,)))
    Dinv = jnp.where(Dn > 0, 1.0 / Dn, 0.0)

    def conv(xin, W, b_row):
        xl = _dot(xin, W, ((1,), (0,)))
        he = Binv * _dot(M, xl, ((0,), (0,)))
        return Dinv * _dot(M, he, ((1,), (0,))) + b_row

    h1 = conv(xi, w1_ref[...], b1_ref[...])
    x1 = jax.nn.relu(h1)
    mu = jnp.mean(x1, axis=1, keepdims=True)
    var = jnp.mean((x1 - mu) ** 2, axis=1, keepdims=True)
    x1 = (x1 - mu) / jnp.sqrt(var + 1e-5) * g_ref[...] + bt_ref[...]

    h2 = conv(x1, w2_ref[...], b2_ref[...])
    out_ref[0] = h2 + xi


def kernel(x, adjacency_matrix, W1, b1, W2, b2, ln_gamma, ln_beta):
    N, C, D = x.shape
    H = W1.shape[1]
    O = W2.shape[1]
    b1r = b1.reshape(1, H)
    b2r = b2.reshape(1, O)
    gr = ln_gamma.reshape(1, H)
    btr = ln_beta.reshape(1, H)

    return pl.pallas_call(
        _graph_body,
        grid=(N,),
        in_specs=[
            pl.BlockSpec((1, C, D), lambda i: (i, 0, 0)),
            pl.BlockSpec((1, C, C), lambda i: (i, 0, 0)),
            pl.BlockSpec((D, H), lambda i: (0, 0)),
            pl.BlockSpec((1, H), lambda i: (0, 0)),
            pl.BlockSpec((H, O), lambda i: (0, 0)),
            pl.BlockSpec((1, O), lambda i: (0, 0)),
            pl.BlockSpec((1, H), lambda i: (0, 0)),
            pl.BlockSpec((1, H), lambda i: (0, 0)),
        ],
        out_specs=pl.BlockSpec((1, C, O), lambda i: (i, 0, 0)),
        out_shape=jax.ShapeDtypeStruct((N, C, O), jnp.float32),
    )(x, adjacency_matrix, W1, b1r, W2, b2r, gr, btr)
